# Initial kernel scaffold; baseline (speedup 1.0000x reference)
import jax
import jax.numpy as jnp
from jax.experimental import pallas as pl


def kernel(x, edge_index, Wl1, bl1, Wr1, Wl2, bl2, Wr2):
    raise NotImplementedError("write your pallas kernel here")


# speedup vs baseline: 3.1080x; 3.1080x over previous
import functools

import jax
import jax.numpy as jnp
from jax import lax
from jax.experimental import pallas as pl
from jax.experimental.pallas import tpu as pltpu
from jax.experimental.pallas import tpu_sc as plsc

N_NODES = 10000
D = 128
N_EDGES = 320000

NC = 2
NS = 16
NW = NC * NS
K = 64
NCHUNK = 160
EDGES_PER_W = NCHUNK * K
E_PAD = NW * EDGES_PER_W
ROWS_PER_TILE = 632
NP = NS * ROWS_PER_TILE
IDXBLK = 16
N_OUTER = NCHUNK // IDXBLK

ROW_BLK = 1000
N_BLK = N_NODES // ROW_BLK


def _make_sc_agg():
  mesh = plsc.VectorSubcoreMesh(core_axis_name="c", subcore_axis_name="s")

  def body(table, src3, dst3, zrows, pout,
           acc, srcv, dstv, rows0, rows1, sem0, sem1):
    c = lax.axis_index("c")
    s = lax.axis_index("s")
    wid = c * NS + s
    stripe = pl.ds(s * ROWS_PER_TILE, ROWS_PER_TILE)

    pltpu.sync_copy(zrows.at[stripe], acc.at[stripe])
    plsc.subcore_barrier()

    def outer(o, carry):
      row0 = wid * NCHUNK + o * IDXBLK
      pltpu.sync_copy(src3.at[pl.ds(row0, IDXBLK)], srcv)
      pltpu.sync_copy(dst3.at[pl.ds(row0, IDXBLK)], dstv)

      def step(t, inner_carry):
        j0 = 2 * t
        j1 = j0 + 1
        g0 = pltpu.async_copy(table.at[srcv.at[j0]], rows0, sem0)
        g1 = pltpu.async_copy(table.at[srcv.at[j1]], rows1, sem1)
        g0.wait()
        pltpu.sync_copy(rows0, acc.at[dstv.at[j0]], add=True)
        g1.wait()
        pltpu.sync_copy(rows1, acc.at[dstv.at[j1]], add=True)
        return inner_carry

      return lax.fori_loop(0, IDXBLK // 2, step, carry)

    lax.fori_loop(0, N_OUTER, outer, 0)
    plsc.subcore_barrier()

    pltpu.sync_copy(acc.at[stripe],
                    pout.at[pl.ds(c * NP + s * ROWS_PER_TILE, ROWS_PER_TILE)])

  return pl.kernel(
      body,
      out_type=[jax.ShapeDtypeStruct((2 * NP, D), jnp.float32)],
      mesh=mesh,
      scratch_types=[
          pltpu.VMEM_SHARED((NP, D), jnp.float32),
          pltpu.VMEM((IDXBLK, K), jnp.int32),
          pltpu.VMEM((IDXBLK, K), jnp.int32),
          pltpu.VMEM((K, D), jnp.float32),
          pltpu.VMEM((K, D), jnp.float32),
          pltpu.SemaphoreType.DMA,
          pltpu.SemaphoreType.DMA,
      ])


def _make_sc_cnt():
  mesh = plsc.VectorSubcoreMesh(core_axis_name="c", subcore_axis_name="s")

  def body(dst3, zrows, ones, cout, cacc, dstv, onesv):
    c = lax.axis_index("c")
    s = lax.axis_index("s")
    wid = c * NS + s
    stripe = pl.ds(s * ROWS_PER_TILE, ROWS_PER_TILE)

    pltpu.sync_copy(zrows.at[stripe], cacc.at[stripe])
    pltpu.sync_copy(ones, onesv)
    plsc.subcore_barrier()

    def outer(o, carry):
      row0 = wid * NCHUNK + o * IDXBLK
      pltpu.sync_copy(dst3.at[pl.ds(row0, IDXBLK)], dstv)

      def step(j, inner_carry):
        pltpu.sync_copy(onesv, cacc.at[dstv.at[j]], add=True)
        return inner_carry

      return lax.fori_loop(0, IDXBLK, step, carry)

    lax.fori_loop(0, N_OUTER, outer, 0)
    plsc.subcore_barrier()

    pltpu.sync_copy(cacc.at[stripe],
                    cout.at[pl.ds(c * NP + s * ROWS_PER_TILE, ROWS_PER_TILE)])

  return pl.kernel(
      body,
      out_type=[jax.ShapeDtypeStruct((2 * NP, D), jnp.float32)],
      mesh=mesh,
      scratch_types=[
          pltpu.VMEM_SHARED((NP, D), jnp.float32),
          pltpu.VMEM((IDXBLK, K), jnp.int32),
          pltpu.VMEM((K, D), jnp.float32),
      ])


def _tc_dense_body(relu, p0, p1, c0, c1, x, wl, wr, bl, o):
  cnt = c0[:, :1] + c1[:, :1]
  mean = (p0[...] + p1[...]) / jnp.maximum(cnt, 1.0)
  acc = lax.dot_general(mean, wl[...], (((1,), (1,)), ((), ())),
                        preferred_element_type=jnp.float32)
  acc = acc + lax.dot_general(x[...], wr[...], (((1,), (1,)), ((), ())),
                              preferred_element_type=jnp.float32)
  acc = acc + bl[...]
  if relu:
    acc = jnp.maximum(acc, 0.0)
  o[...] = acc


def _make_tc_dense(relu: bool, interpret: bool = False):
  row = lambda i: (i, 0)
  fixed = lambda i: (0, 0)
  return pl.pallas_call(
      functools.partial(_tc_dense_body, relu),
      grid=(N_BLK,),
      in_specs=[
          pl.BlockSpec((ROW_BLK, D), row),
          pl.BlockSpec((ROW_BLK, D), row),
          pl.BlockSpec((ROW_BLK, D), row),
          pl.BlockSpec((ROW_BLK, D), row),
          pl.BlockSpec((ROW_BLK, D), row),
          pl.BlockSpec((D, D), fixed),
          pl.BlockSpec((D, D), fixed),
          pl.BlockSpec((1, D), fixed),
      ],
      out_specs=pl.BlockSpec((ROW_BLK, D), row),
      out_shape=jax.ShapeDtypeStruct((N_NODES, D), jnp.float32),
      interpret=interpret,
  )


_sc_agg_cache = functools.cache(_make_sc_agg)
_sc_cnt_cache = functools.cache(_make_sc_cnt)
_dense_relu = _make_tc_dense(relu=True)
_dense_lin = _make_tc_dense(relu=False)


def kernel(x, edge_index, Wl1, bl1, Wr1, Wl2, bl2, Wr2):
  src = edge_index[0].astype(jnp.int32)
  dst = edge_index[1].astype(jnp.int32)
  pad = E_PAD - N_EDGES
  src3 = jnp.concatenate([src, jnp.zeros((pad,), jnp.int32)]).reshape(NW * NCHUNK, K)
  dst3 = jnp.concatenate(
      [dst, N_NODES + (jnp.arange(pad, dtype=jnp.int32) % 16)]
  ).reshape(NW * NCHUNK, K)
  zrows = jnp.zeros((NP, D), jnp.float32)
  ones = jnp.ones((K, D), jnp.float32)

  (cnt,) = _sc_cnt_cache()(dst3, zrows, ones)
  (pout,) = _sc_agg_cache()(x, src3, dst3, zrows)
  c0 = cnt[:N_NODES]
  c1 = cnt[NP:NP + N_NODES]
  h = _dense_relu(pout[:N_NODES], pout[NP:NP + N_NODES], c0, c1, x,
                  Wl1, Wr1, bl1.reshape(1, D))
  (qout,) = _sc_agg_cache()(h, src3, dst3, zrows)
  out = _dense_lin(qout[:N_NODES], qout[NP:NP + N_NODES], c0, c1, h,
                   Wl2, Wr2, bl2.reshape(1, D))
  return out
